# Initial kernel scaffold; baseline (speedup 1.0000x reference)
import jax
import jax.numpy as jnp
from jax.experimental import pallas as pl


def kernel(z, embeddings):
    raise NotImplementedError("write your pallas kernel here")


# speedup vs baseline: 1.0107x; 1.0107x over previous
import functools

import jax
import jax.numpy as jnp
from jax import lax
from jax.experimental import pallas as pl
from jax.experimental.pallas import tpu as pltpu
from jax.experimental.pallas import tpu_sc as plsc

_D = 256
_K = 8192
_KWIN = 1664
_KPAD = 5 * _KWIN
_BN = 512
_N = 16384
_COMMIT = 0.25


def _dist_body(x_ref, e_ref, idx_ref, dval_ref, acc, aidx, aval):
    kt = pl.program_id(0)
    nt = pl.program_id(1)
    rows = pl.ds(nt * _BN, _BN)
    x = x_ref[...]
    e = e_ref[...]
    xb = (2.0 * x).astype(jnp.bfloat16)
    eb = e.astype(jnp.bfloat16)
    s2 = lax.dot_general(
        xb, eb, (((1,), (0,)), ((), ())), preferred_element_type=jnp.float32
    )
    xsq = jnp.sum(x * x, axis=1, keepdims=True)
    esq = jnp.sum(e * e, axis=0, keepdims=True)
    lane = lax.broadcasted_iota(jnp.int32, (1, _KWIN), 1) + kt * _KWIN
    esq = jnp.where(lane >= _K, jnp.inf, esq)
    d = (xsq - s2) + esq
    dmin_t = jnp.min(d, axis=1, keepdims=True)
    amin_t = jnp.argmin(d, axis=1).astype(jnp.int32)[:, None] + kt * _KWIN

    @pl.when(kt == 0)
    def _():
        acc[rows] = jnp.full((_BN, 1), jnp.inf, jnp.float32)
        aidx[rows] = jnp.full((_BN, 1), jnp.int32(2**30))
        aval[rows] = jnp.full((_BN, 1), jnp.inf, jnp.float32)

    a = acc[rows]
    ai = aidx[rows]
    better = (dmin_t < a) | ((dmin_t == a) & (amin_t < ai))
    acc[rows] = (
        jnp.where(better, dmin_t, a).astype(jnp.bfloat16).astype(jnp.float32)
    )
    aidx[rows] = jnp.where(better, amin_t, ai)
    aval[rows] = jnp.where(better, dmin_t, aval[rows])

    @pl.when(kt == pl.num_programs(0) - 1)
    def _():
        idx_ref[0] = aidx[rows]
        dval_ref[0] = aval[rows]


def _dist_argmin(flat, e_pad):
    nt = _N // _BN
    idx3, dval3 = pl.pallas_call(
        _dist_body,
        grid=(5, nt),
        in_specs=[
            pl.BlockSpec((_BN, _D), lambda k, i: (i, 0)),
            pl.BlockSpec((_D, _KWIN), lambda k, i: (0, k)),
        ],
        out_specs=[
            pl.BlockSpec((1, _BN, 1), lambda k, i: (i, 0, 0)),
            pl.BlockSpec((1, _BN, 1), lambda k, i: (i, 0, 0)),
        ],
        out_shape=[
            jax.ShapeDtypeStruct((nt, _BN, 1), jnp.int32),
            jax.ShapeDtypeStruct((nt, _BN, 1), jnp.float32),
        ],
        scratch_shapes=[
            pltpu.VMEM((_N, 1), jnp.float32),
            pltpu.VMEM((_N, 1), jnp.int32),
            pltpu.VMEM((_N, 1), jnp.float32),
        ],
        compiler_params=pltpu.CompilerParams(
            dimension_semantics=("arbitrary", "arbitrary"),
        ),
    )(flat, e_pad)
    return idx3.reshape(_N), dval3.reshape(_N)


_CHUNK = 128
_NCHUNK = 4


def _sc_stage(table, idx, zeros_k, ones_c):
    info = plsc.get_sparse_core_info()
    nc, ns = info.num_cores, info.num_subcores
    nw = nc * ns
    per_w = _N // nw
    mesh = plsc.VectorSubcoreMesh(core_axis_name="c", subcore_axis_name="s")

    @functools.partial(
        pl.kernel,
        mesh=mesh,
        out_type=[
            jax.ShapeDtypeStruct((_N, _D), jnp.float32),
            jax.ShapeDtypeStruct((nc, _K), jnp.float32),
        ],
        scratch_types=[
            pltpu.VMEM((_NCHUNK, _CHUNK), jnp.int32),
            pltpu.VMEM((_CHUNK, _D), jnp.float32),
            pltpu.VMEM((_CHUNK,), jnp.float32),
            pltpu.VMEM_SHARED((_K,), jnp.float32),
            pltpu.SemaphoreType.DMA,
        ],
    )
    def k(table_hbm, idx_hbm, zeros_hbm, ones_hbm, out_hbm, cnt_hbm,
          idx_v, rows_v, ones_v, cnt_sh, sem):
        cid = lax.axis_index("c")
        sid = lax.axis_index("s")
        wid = sid * nc + cid
        base = wid * per_w

        @pl.when(sid == 0)
        def _():
            pltpu.sync_copy(zeros_hbm, cnt_sh)

        pltpu.sync_copy(ones_hbm, ones_v)
        for c in range(_NCHUNK):
            pltpu.sync_copy(
                idx_hbm.at[pl.ds(base + c * _CHUNK, _CHUNK)], idx_v.at[c]
            )
        plsc.subcore_barrier()

        for c in range(_NCHUNK):
            pltpu.async_copy(table_hbm.at[idx_v.at[c]], rows_v, sem).wait()
            pltpu.sync_copy(
                rows_v, out_hbm.at[pl.ds(base + c * _CHUNK, _CHUNK)]
            )
            pltpu.sync_copy(ones_v, cnt_sh.at[idx_v.at[c]], add=True)

        plsc.subcore_barrier()

        @pl.when(sid == 0)
        def _():
            pltpu.sync_copy(cnt_sh, cnt_hbm.at[cid])

    return k(table, idx, zeros_k, ones_c)


def _final_body(dval_ref, cnt_ref, loss_ref, perp_ref):
    s = jnp.sum(dval_ref[...])
    loss_ref[0, 0] = s * (_COMMIT / (_N * _D))
    c = cnt_ref[0, :] + cnt_ref[1, :]
    avg = c * (1.0 / _N)
    ent = jnp.sum(avg * jnp.log(avg + 1e-10))
    perp_ref[0, 0] = jnp.exp(-ent)


def _finalize(dval, counts):
    loss, perp = pl.pallas_call(
        _final_body,
        in_specs=[
            pl.BlockSpec(memory_space=pltpu.VMEM),
            pl.BlockSpec(memory_space=pltpu.VMEM),
        ],
        out_specs=[
            pl.BlockSpec(memory_space=pltpu.SMEM),
            pl.BlockSpec(memory_space=pltpu.SMEM),
        ],
        out_shape=[
            jax.ShapeDtypeStruct((1, 1), jnp.float32),
            jax.ShapeDtypeStruct((1, 1), jnp.float32),
        ],
    )(dval.reshape(128, 128), counts)
    return loss.reshape(()), perp.reshape(())


def kernel(z, embeddings):
    flat = z.reshape(-1, z.shape[-1])
    e_pad = jnp.pad(embeddings, ((0, 0), (0, _KPAD - _K)))
    idx, dval = _dist_argmin(flat, e_pad)
    quantized, counts = _sc_stage(
        embeddings.T,
        idx,
        jnp.zeros((_K,), jnp.float32),
        jnp.ones((_CHUNK,), jnp.float32),
    )
    loss, perp = _finalize(dval, counts)
    q = quantized.reshape(z.shape)
    quantized_st = z + lax.stop_gradient(q - z)
    return quantized_st, loss, perp, idx.reshape(z.shape[:-1])


# speedup vs baseline: 1.0681x; 1.0568x over previous
import functools

import jax
import jax.numpy as jnp
from jax import lax
from jax.experimental import pallas as pl
from jax.experimental.pallas import tpu as pltpu
from jax.experimental.pallas import tpu_sc as plsc

_D = 256
_K = 8192
_KWIN = 1664
_KPAD = 5 * _KWIN
_BN = 512
_N = 16384
_COMMIT = 0.25


def _dist_body(x_ref, e_ref, idx_ref, dval_ref, acc, aidx, aval, esq_s, eb_s):
    kt = pl.program_id(0)
    nt = pl.program_id(1)
    rows = pl.ds(nt * _BN, _BN)
    x = x_ref[...]

    @pl.when(nt == 0)
    def _():
        e = e_ref[...]
        eb_s[...] = e.astype(jnp.bfloat16)
        esq = jnp.sum(e * e, axis=0, keepdims=True)
        lane = lax.broadcasted_iota(jnp.int32, (1, _KWIN), 1) + kt * _KWIN
        esq_s[...] = jnp.where(lane >= _K, jnp.inf, esq)

    xb = (2.0 * x).astype(jnp.bfloat16)
    s2 = lax.dot_general(
        xb, eb_s[...], (((1,), (0,)), ((), ())),
        preferred_element_type=jnp.float32,
    )
    xsq = jnp.sum(x * x, axis=1, keepdims=True)
    d = (xsq - s2) + esq_s[...]
    dmin_t = jnp.min(d, axis=1, keepdims=True)
    amin_t = jnp.argmin(d, axis=1).astype(jnp.int32)[:, None] + kt * _KWIN

    @pl.when(kt == 0)
    def _():
        acc[rows] = jnp.full((_BN, 1), jnp.inf, jnp.float32)
        aidx[rows] = jnp.full((_BN, 1), jnp.int32(2**30))
        aval[rows] = jnp.full((_BN, 1), jnp.inf, jnp.float32)

    a = acc[rows]
    ai = aidx[rows]
    better = (dmin_t < a) | ((dmin_t == a) & (amin_t < ai))
    acc[rows] = (
        jnp.where(better, dmin_t, a).astype(jnp.bfloat16).astype(jnp.float32)
    )
    aidx[rows] = jnp.where(better, amin_t, ai)
    aval[rows] = jnp.where(better, dmin_t, aval[rows])

    @pl.when(kt == pl.num_programs(0) - 1)
    def _():
        idx_ref[0] = aidx[rows]
        dval_ref[0] = aval[rows]


def _dist_argmin(flat, e_pad):
    nt = _N // _BN
    idx3, dval3 = pl.pallas_call(
        _dist_body,
        grid=(5, nt),
        in_specs=[
            pl.BlockSpec((_BN, _D), lambda k, i: (i, 0)),
            pl.BlockSpec((_D, _KWIN), lambda k, i: (0, k)),
        ],
        out_specs=[
            pl.BlockSpec((1, _BN, 1), lambda k, i: (i, 0, 0)),
            pl.BlockSpec((1, _BN, 1), lambda k, i: (i, 0, 0)),
        ],
        out_shape=[
            jax.ShapeDtypeStruct((nt, _BN, 1), jnp.int32),
            jax.ShapeDtypeStruct((nt, _BN, 1), jnp.float32),
        ],
        scratch_shapes=[
            pltpu.VMEM((_N, 1), jnp.float32),
            pltpu.VMEM((_N, 1), jnp.int32),
            pltpu.VMEM((_N, 1), jnp.float32),
            pltpu.VMEM((1, _KWIN), jnp.float32),
            pltpu.VMEM((_D, _KWIN), jnp.bfloat16),
        ],
        compiler_params=pltpu.CompilerParams(
            dimension_semantics=("arbitrary", "arbitrary"),
        ),
    )(flat, e_pad)
    return idx3.reshape(_N), dval3.reshape(_N)


_CHUNK = 128
_NCHUNK = 4


def _sc_stage(table, idx, zeros_k, ones_c):
    info = plsc.get_sparse_core_info()
    nc, ns = info.num_cores, info.num_subcores
    nw = nc * ns
    per_w = _N // nw
    mesh = plsc.VectorSubcoreMesh(core_axis_name="c", subcore_axis_name="s")

    @functools.partial(
        pl.kernel,
        mesh=mesh,
        out_type=[
            jax.ShapeDtypeStruct((_N, _D), jnp.float32),
            jax.ShapeDtypeStruct((nc, _K), jnp.float32),
        ],
        scratch_types=[
            pltpu.VMEM((_NCHUNK, _CHUNK), jnp.int32),
            pltpu.VMEM((_CHUNK, _D), jnp.float32),
            pltpu.VMEM((_CHUNK,), jnp.float32),
            pltpu.VMEM_SHARED((_K,), jnp.float32),
            pltpu.SemaphoreType.DMA,
        ],
    )
    def k(table_hbm, idx_hbm, zeros_hbm, ones_hbm, out_hbm, cnt_hbm,
          idx_v, rows_v, ones_v, cnt_sh, sem):
        cid = lax.axis_index("c")
        sid = lax.axis_index("s")
        wid = sid * nc + cid
        base = wid * per_w

        @pl.when(sid == 0)
        def _():
            pltpu.sync_copy(zeros_hbm, cnt_sh)

        pltpu.sync_copy(ones_hbm, ones_v)
        for c in range(_NCHUNK):
            pltpu.sync_copy(
                idx_hbm.at[pl.ds(base + c * _CHUNK, _CHUNK)], idx_v.at[c]
            )
        plsc.subcore_barrier()

        for c in range(_NCHUNK):
            pltpu.async_copy(table_hbm.at[idx_v.at[c]], rows_v, sem).wait()
            pltpu.sync_copy(
                rows_v, out_hbm.at[pl.ds(base + c * _CHUNK, _CHUNK)]
            )
            pltpu.sync_copy(ones_v, cnt_sh.at[idx_v.at[c]], add=True)

        plsc.subcore_barrier()

        @pl.when(sid == 0)
        def _():
            pltpu.sync_copy(cnt_sh, cnt_hbm.at[cid])

    return k(table, idx, zeros_k, ones_c)


def _final_body(dval_ref, cnt_ref, loss_ref, perp_ref):
    s = jnp.sum(dval_ref[...])
    loss_ref[0, 0] = s * (_COMMIT / (_N * _D))
    c = cnt_ref[0, :] + cnt_ref[1, :]
    avg = c * (1.0 / _N)
    ent = jnp.sum(avg * jnp.log(avg + 1e-10))
    perp_ref[0, 0] = jnp.exp(-ent)


def _finalize(dval, counts):
    loss, perp = pl.pallas_call(
        _final_body,
        in_specs=[
            pl.BlockSpec(memory_space=pltpu.VMEM),
            pl.BlockSpec(memory_space=pltpu.VMEM),
        ],
        out_specs=[
            pl.BlockSpec(memory_space=pltpu.SMEM),
            pl.BlockSpec(memory_space=pltpu.SMEM),
        ],
        out_shape=[
            jax.ShapeDtypeStruct((1, 1), jnp.float32),
            jax.ShapeDtypeStruct((1, 1), jnp.float32),
        ],
    )(dval.reshape(128, 128), counts)
    return loss.reshape(()), perp.reshape(())


def kernel(z, embeddings):
    flat = z.reshape(-1, z.shape[-1])
    e_pad = jnp.pad(embeddings, ((0, 0), (0, _KPAD - _K)))
    idx, dval = _dist_argmin(flat, e_pad)
    quantized, counts = _sc_stage(
        embeddings.T,
        idx,
        jnp.zeros((_K,), jnp.float32),
        jnp.ones((_CHUNK,), jnp.float32),
    )
    loss, perp = _finalize(dval, counts)
    q = quantized.reshape(z.shape)
    quantized_st = z + lax.stop_gradient(q - z)
    return quantized_st, loss, perp, idx.reshape(z.shape[:-1])
